# baseline (speedup 1.0000x reference)
import jax
import jax.numpy as jnp
from jax.experimental import pallas as pl
from jax.experimental.pallas import tpu as pltpu


def _pos5_kernel(xe_ref, ye_ref, ze_ref, out_ref):
    x = xe_ref[...]
    y = ye_ref[...]
    z = ze_ref[...]
    out_ref[...] = (
        x[None, :, :, None, None] + y[None, :, None, :, None] + z[None, :, None, None, :]
    )


def kernel(features, x_embed, y_embed, z_embed):
    b = features.shape[0]
    nx, ny, nz = features.shape[2], features.shape[3], features.shape[4]
    d = x_embed.shape[1]

    xeT = x_embed[:nx].T
    yeT = y_embed[:ny].T
    zeT = z_embed[:nz].T

    D = 16
    grid = (b, d // D)

    out = pl.pallas_call(
        _pos5_kernel,
        grid=grid,
        in_specs=[
            pl.BlockSpec((D, nx), lambda i, j: (j, 0)),
            pl.BlockSpec((D, ny), lambda i, j: (j, 0)),
            pl.BlockSpec((D, nz), lambda i, j: (j, 0)),
        ],
        out_specs=pl.BlockSpec((1, D, nx, ny, nz), lambda i, j: (i, j, 0, 0, 0)),
        out_shape=jax.ShapeDtypeStruct((b, d, nx, ny, nz), jnp.float32),
        compiler_params=pltpu.CompilerParams(
            dimension_semantics=("parallel", "arbitrary"),
        ),
    )(xeT, yeT, zeT)

    return out
